# baseline (speedup 1.0000x reference)
import functools

import jax
import jax.numpy as jnp
from jax import lax
from jax.experimental import pallas as pl
from jax.experimental.pallas import tpu as pltpu
from jax.experimental.pallas import tpu_sc as plsc

N = 10000
D = 128
T = 64
G = 64

ACCROWS = 10112
C = 64
NCHUNK = 168
NBUF = 3
NSTAGE = 7
NW = 32
EP = NW * NCHUNK * C
TROWS = ACCROWS // 16

_mesh = plsc.VectorSubcoreMesh(core_axis_name="c", subcore_axis_name="s")


def _make_agg(d):
    @functools.partial(
        pl.kernel,
        out_type=jax.ShapeDtypeStruct((2 * ACCROWS, d), jnp.float32),
        mesh=_mesh,
        scratch_types=[
            pltpu.VMEM((NCHUNK // NSTAGE, C), jnp.int32),
            pltpu.VMEM((NCHUNK // NSTAGE, C), jnp.int32),
            pltpu.VMEM_SHARED((ACCROWS, d), jnp.float32),
        ] + [pltpu.VMEM((C, d), jnp.float32) for _ in range(NBUF)]
          + [pltpu.SemaphoreType.DMA] * (2 * NBUF),
    )
    def agg(table_hbm, src_hbm, dst_hbm, out_hbm, src_v, dst_v, acc, *rest):
        ring = rest[:NBUF]
        gsem = rest[NBUF:2 * NBUF]
        ssem = rest[2 * NBUF:3 * NBUF]
        cid = lax.axis_index("c")
        sid = lax.axis_index("s")
        wid = cid * 16 + sid
        stg = NCHUNK // NSTAGE

        @pl.loop(0, C)
        def _(r):
            @pl.loop(0, d, step=16)
            def _(cc):
                ring[0][r, pl.ds(cc, 16)] = jnp.zeros((16,), jnp.float32)

        r0 = sid * TROWS
        nfull = TROWS // C
        for k in range(nfull):
            pltpu.sync_copy(ring[0], acc.at[pl.ds(r0 + k * C, C)])
        rem = TROWS - nfull * C
        if rem:
            pltpu.sync_copy(ring[0].at[pl.ds(0, rem)],
                            acc.at[pl.ds(r0 + nfull * C, rem)])

        plsc.subcore_barrier()

        def wait_gather(j, b):
            pltpu.make_async_copy(table_hbm.at[src_v.at[j]], ring[b],
                                  gsem[b]).wait()

        def wait_scatter(j, b):
            pltpu.make_async_copy(ring[b], acc.at[dst_v.at[j]],
                                  ssem[b]).wait()

        for p in range(NSTAGE):
            pltpu.sync_copy(src_hbm.at[wid, pl.ds(p * stg, stg)], src_v)
            pltpu.sync_copy(dst_hbm.at[wid, pl.ds(p * stg, stg)], dst_v)

            @pl.loop(0, stg, step=NBUF)
            def _(j):
                for b in range(NBUF):
                    @pl.when(j >= NBUF)
                    def _():
                        wait_scatter(j - NBUF + b, b)

                    pltpu.async_copy(table_hbm.at[src_v.at[j + b]], ring[b],
                                     gsem[b])
                for b in range(NBUF):
                    wait_gather(j + b, b)
                    pltpu.async_copy(ring[b], acc.at[dst_v.at[j + b]],
                                     ssem[b], add=True)

            for b in range(NBUF):
                wait_scatter(stg - NBUF + b, b)

        plsc.subcore_barrier()

        oo = pl.multiple_of(cid * ACCROWS + r0, 8)
        pltpu.sync_copy(acc.at[pl.ds(r0, TROWS)],
                        out_hbm.at[pl.ds(oo, TROWS)])

    return agg


@functools.partial(
    pl.kernel,
    out_type=jax.ShapeDtypeStruct((2 * ACCROWS, D), jnp.float32),
    mesh=_mesh,
    scratch_types=[
        pltpu.VMEM((NCHUNK, C), jnp.int32),
        pltpu.VMEM((C, D), jnp.float32),
        pltpu.VMEM_SHARED((ACCROWS, D), jnp.float32),
    ],
)
def _cnt_kernel(dst_hbm, out_hbm, dst_v, ones_v, acc):
    cid = lax.axis_index("c")
    sid = lax.axis_index("s")
    wid = cid * 16 + sid
    pltpu.sync_copy(dst_hbm.at[wid], dst_v)

    @pl.loop(0, C)
    def _(r):
        @pl.loop(0, D, step=16)
        def _(cc):
            ones_v[r, pl.ds(cc, 16)] = jnp.zeros((16,), jnp.float32)

    r0 = sid * TROWS
    nfull = TROWS // C
    for k in range(nfull):
        pltpu.sync_copy(ones_v, acc.at[pl.ds(r0 + k * C, C)])
    rem = TROWS - nfull * C
    if rem:
        pltpu.sync_copy(ones_v.at[pl.ds(0, rem)],
                        acc.at[pl.ds(r0 + nfull * C, rem)])

    @pl.loop(0, C)
    def _(r):
        @pl.loop(0, D, step=16)
        def _(cc):
            ones_v[r, pl.ds(cc, 16)] = jnp.ones((16,), jnp.float32)

    plsc.subcore_barrier()

    @pl.loop(0, NCHUNK)
    def _(j):
        pltpu.sync_copy(ones_v, acc.at[dst_v.at[j]], add=True)

    plsc.subcore_barrier()
    oo = pl.multiple_of(cid * ACCROWS + r0, 8)
    pltpu.sync_copy(acc.at[pl.ds(r0, TROWS)], out_hbm.at[pl.ds(oo, TROWS)])


_agg128 = _make_agg(D)


def _mean_from_partials(p_ref, cnt_ref):
    psum = p_ref[0:N, :] + p_ref[ACCROWS:ACCROWS + N, :]
    cnt = cnt_ref[0:N, 0:1] + cnt_ref[ACCROWS:ACCROWS + N, 0:1]
    return psum / jnp.maximum(cnt, 1.0)


def _dense_body(p_ref, cnt_ref, h_ref, wl_ref, wr_ref, b_ref, g_ref, be_ref,
                o_ref):
    mean = _mean_from_partials(p_ref, cnt_ref)
    z = (jnp.dot(mean, wl_ref[...], preferred_element_type=jnp.float32)
         + jnp.dot(h_ref[...], wr_ref[...], preferred_element_type=jnp.float32)
         + b_ref[...])
    mu = jnp.mean(z, axis=0, keepdims=True)
    var = jnp.mean((z - mu) * (z - mu), axis=0, keepdims=True)
    zn = (z - mu) / jnp.sqrt(var + 1e-5) * g_ref[...] + be_ref[...]
    o_ref[...] = jnp.maximum(zn, 0.0)


def _final_body(p_ref, cnt_ref, h_ref, batch_ref, wlo_ref, wro_ref, bo_ref,
                o_ref):
    s = _mean_from_partials(p_ref, cnt_ref)
    gi = lax.broadcasted_iota(jnp.int32, (G, N), 0)
    bm = (batch_ref[0:1, :] == gi).astype(jnp.float32)
    cg = jnp.sum(bm, axis=1, keepdims=True)
    pw = bm * jnp.where(cg > 0, 1.0 / jnp.maximum(cg, 1.0), 0.0)
    pooled_s = jnp.dot(pw, s, preferred_element_type=jnp.float32)
    pooled_h = jnp.dot(pw, h_ref[...], preferred_element_type=jnp.float32)
    o = (jnp.dot(pooled_s, wlo_ref[...], preferred_element_type=jnp.float32)
         + jnp.dot(pooled_h, wro_ref[...], preferred_element_type=jnp.float32)
         + bo_ref[...])
    o_ref[...] = jnp.where(cg > 0, o, 0.0)


def _dense(p, cntp, h, wl, wr, b, g, be):
    return pl.pallas_call(
        _dense_body,
        out_shape=jax.ShapeDtypeStruct((N, D), jnp.float32),
    )(p, cntp, h, wl, wr, b.reshape(1, -1), g.reshape(1, -1),
      be.reshape(1, -1))


def _final(p, cntp, h, batch2d, wlo, wro, bo):
    return pl.pallas_call(
        _final_body,
        out_shape=jax.ShapeDtypeStruct((G, T), jnp.float32),
    )(p, cntp, h, batch2d, wlo, wro, bo.reshape(1, -1))


def kernel(x, edge_index, batch, Wl0, Wr0, b0, g1, be1, Wl1, Wr1, b1, g2, be2,
           Wl2, Wr2, b2, g3, be3, Wlo, Wro, bo):
    e = edge_index.shape[1]
    pad = EP - e
    pidx = jnp.arange(pad, dtype=jnp.int32)
    src = jnp.concatenate(
        [edge_index[0], pidx % N]).reshape(NW, NCHUNK, C)
    dst = jnp.concatenate(
        [edge_index[1], N + pidx % (ACCROWS - N)]).reshape(NW, NCHUNK, C)

    cntp = _cnt_kernel(dst)
    p0 = _agg128(x, src, dst)
    h1 = _dense(p0, cntp, x, Wl0, Wr0, b0, g1, be1)
    p1 = _agg128(h1, src, dst)
    h2 = _dense(p1, cntp, h1, Wl1, Wr1, b1, g2, be2)
    p2 = _agg128(h2, src, dst)
    h3 = _dense(p2, cntp, h2, Wl2, Wr2, b2, g3, be3)
    p3 = _agg128(h3, src, dst)
    return _final(p3, cntp, h3, batch.reshape(1, N), Wlo, Wro, bo)


# speedup vs baseline: 1.0176x; 1.0176x over previous
import functools

import jax
import jax.numpy as jnp
from jax import lax
from jax.experimental import pallas as pl
from jax.experimental.pallas import tpu as pltpu
from jax.experimental.pallas import tpu_sc as plsc

N = 10000
D = 128
T = 64
G = 64

ACCROWS = 10112
C = 128
NCHUNK = 80
NBUF = 2
NSTAGE = 2
NW = 32
EP = NW * NCHUNK * C
TROWS = ACCROWS // 16

_mesh = plsc.VectorSubcoreMesh(core_axis_name="c", subcore_axis_name="s")


def _make_agg(d):
    @functools.partial(
        pl.kernel,
        out_type=jax.ShapeDtypeStruct((2 * ACCROWS, d), jnp.float32),
        mesh=_mesh,
        scratch_types=[
            pltpu.VMEM((NCHUNK // NSTAGE, C), jnp.int32),
            pltpu.VMEM((NCHUNK // NSTAGE, C), jnp.int32),
            pltpu.VMEM_SHARED((ACCROWS, d), jnp.float32),
        ] + [pltpu.VMEM((C, d), jnp.float32) for _ in range(NBUF)]
          + [pltpu.SemaphoreType.DMA] * (2 * NBUF),
    )
    def agg(table_hbm, src_hbm, dst_hbm, out_hbm, src_v, dst_v, acc, *rest):
        ring = rest[:NBUF]
        gsem = rest[NBUF:2 * NBUF]
        ssem = rest[2 * NBUF:3 * NBUF]
        cid = lax.axis_index("c")
        sid = lax.axis_index("s")
        wid = cid * 16 + sid
        stg = NCHUNK // NSTAGE

        @pl.loop(0, C)
        def _(r):
            @pl.loop(0, d, step=16)
            def _(cc):
                ring[0][r, pl.ds(cc, 16)] = jnp.zeros((16,), jnp.float32)

        r0 = sid * TROWS
        nfull = TROWS // C
        for k in range(nfull):
            pltpu.sync_copy(ring[0], acc.at[pl.ds(r0 + k * C, C)])
        rem = TROWS - nfull * C
        if rem:
            pltpu.sync_copy(ring[0].at[pl.ds(0, rem)],
                            acc.at[pl.ds(r0 + nfull * C, rem)])

        plsc.subcore_barrier()

        def wait_gather(j, b):
            pltpu.make_async_copy(table_hbm.at[src_v.at[j]], ring[b],
                                  gsem[b]).wait()

        def wait_scatter(j, b):
            pltpu.make_async_copy(ring[b], acc.at[dst_v.at[j]],
                                  ssem[b]).wait()

        for p in range(NSTAGE):
            pltpu.sync_copy(src_hbm.at[wid, pl.ds(p * stg, stg)], src_v)
            pltpu.sync_copy(dst_hbm.at[wid, pl.ds(p * stg, stg)], dst_v)

            @pl.loop(0, stg, step=NBUF)
            def _(j):
                for b in range(NBUF):
                    @pl.when(j >= NBUF)
                    def _():
                        wait_scatter(j - NBUF + b, b)

                    pltpu.async_copy(table_hbm.at[src_v.at[j + b]], ring[b],
                                     gsem[b])
                for b in range(NBUF):
                    wait_gather(j + b, b)
                    pltpu.async_copy(ring[b], acc.at[dst_v.at[j + b]],
                                     ssem[b], add=True)

            for b in range(NBUF):
                wait_scatter(stg - NBUF + b, b)

        plsc.subcore_barrier()

        oo = pl.multiple_of(cid * ACCROWS + r0, 8)
        pltpu.sync_copy(acc.at[pl.ds(r0, TROWS)],
                        out_hbm.at[pl.ds(oo, TROWS)])

    return agg


@functools.partial(
    pl.kernel,
    out_type=jax.ShapeDtypeStruct((2 * ACCROWS, D), jnp.float32),
    mesh=_mesh,
    scratch_types=[
        pltpu.VMEM((NCHUNK, C), jnp.int32),
        pltpu.VMEM((C, D), jnp.float32),
        pltpu.VMEM_SHARED((ACCROWS, D), jnp.float32),
    ],
)
def _cnt_kernel(dst_hbm, out_hbm, dst_v, ones_v, acc):
    cid = lax.axis_index("c")
    sid = lax.axis_index("s")
    wid = cid * 16 + sid
    pltpu.sync_copy(dst_hbm.at[wid], dst_v)

    @pl.loop(0, C)
    def _(r):
        @pl.loop(0, D, step=16)
        def _(cc):
            ones_v[r, pl.ds(cc, 16)] = jnp.zeros((16,), jnp.float32)

    r0 = sid * TROWS
    nfull = TROWS // C
    for k in range(nfull):
        pltpu.sync_copy(ones_v, acc.at[pl.ds(r0 + k * C, C)])
    rem = TROWS - nfull * C
    if rem:
        pltpu.sync_copy(ones_v.at[pl.ds(0, rem)],
                        acc.at[pl.ds(r0 + nfull * C, rem)])

    @pl.loop(0, C)
    def _(r):
        @pl.loop(0, D, step=16)
        def _(cc):
            ones_v[r, pl.ds(cc, 16)] = jnp.ones((16,), jnp.float32)

    plsc.subcore_barrier()

    @pl.loop(0, NCHUNK)
    def _(j):
        pltpu.sync_copy(ones_v, acc.at[dst_v.at[j]], add=True)

    plsc.subcore_barrier()
    oo = pl.multiple_of(cid * ACCROWS + r0, 8)
    pltpu.sync_copy(acc.at[pl.ds(r0, TROWS)], out_hbm.at[pl.ds(oo, TROWS)])


_agg128 = _make_agg(D)


def _mean_from_partials(p_ref, cnt_ref):
    psum = p_ref[0:N, :] + p_ref[ACCROWS:ACCROWS + N, :]
    cnt = cnt_ref[0:N, 0:1] + cnt_ref[ACCROWS:ACCROWS + N, 0:1]
    return psum / jnp.maximum(cnt, 1.0)


def _dense_body(p_ref, cnt_ref, h_ref, wl_ref, wr_ref, b_ref, g_ref, be_ref,
                o_ref):
    mean = _mean_from_partials(p_ref, cnt_ref)
    z = (jnp.dot(mean, wl_ref[...], preferred_element_type=jnp.float32)
         + jnp.dot(h_ref[...], wr_ref[...], preferred_element_type=jnp.float32)
         + b_ref[...])
    mu = jnp.mean(z, axis=0, keepdims=True)
    var = jnp.mean((z - mu) * (z - mu), axis=0, keepdims=True)
    zn = (z - mu) / jnp.sqrt(var + 1e-5) * g_ref[...] + be_ref[...]
    o_ref[...] = jnp.maximum(zn, 0.0)


def _final_body(p_ref, cnt_ref, h_ref, batch_ref, wlo_ref, wro_ref, bo_ref,
                o_ref):
    s = _mean_from_partials(p_ref, cnt_ref)
    gi = lax.broadcasted_iota(jnp.int32, (G, N), 0)
    bm = (batch_ref[0:1, :] == gi).astype(jnp.float32)
    cg = jnp.sum(bm, axis=1, keepdims=True)
    pw = bm * jnp.where(cg > 0, 1.0 / jnp.maximum(cg, 1.0), 0.0)
    pooled_s = jnp.dot(pw, s, preferred_element_type=jnp.float32)
    pooled_h = jnp.dot(pw, h_ref[...], preferred_element_type=jnp.float32)
    o = (jnp.dot(pooled_s, wlo_ref[...], preferred_element_type=jnp.float32)
         + jnp.dot(pooled_h, wro_ref[...], preferred_element_type=jnp.float32)
         + bo_ref[...])
    o_ref[...] = jnp.where(cg > 0, o, 0.0)


def _dense(p, cntp, h, wl, wr, b, g, be):
    return pl.pallas_call(
        _dense_body,
        out_shape=jax.ShapeDtypeStruct((N, D), jnp.float32),
    )(p, cntp, h, wl, wr, b.reshape(1, -1), g.reshape(1, -1),
      be.reshape(1, -1))


def _final(p, cntp, h, batch2d, wlo, wro, bo):
    return pl.pallas_call(
        _final_body,
        out_shape=jax.ShapeDtypeStruct((G, T), jnp.float32),
    )(p, cntp, h, batch2d, wlo, wro, bo.reshape(1, -1))


def kernel(x, edge_index, batch, Wl0, Wr0, b0, g1, be1, Wl1, Wr1, b1, g2, be2,
           Wl2, Wr2, b2, g3, be3, Wlo, Wro, bo):
    e = edge_index.shape[1]
    pad = EP - e
    pidx = jnp.arange(pad, dtype=jnp.int32)
    src = jnp.concatenate(
        [edge_index[0], pidx % N]).reshape(NW, NCHUNK, C)
    dst = jnp.concatenate(
        [edge_index[1], N + pidx % (ACCROWS - N)]).reshape(NW, NCHUNK, C)

    cntp = _cnt_kernel(dst)
    p0 = _agg128(x, src, dst)
    h1 = _dense(p0, cntp, x, Wl0, Wr0, b0, g1, be1)
    p1 = _agg128(h1, src, dst)
    h2 = _dense(p1, cntp, h1, Wl1, Wr1, b1, g2, be2)
    p2 = _agg128(h2, src, dst)
    h3 = _dense(p2, cntp, h2, Wl2, Wr2, b2, g3, be3)
    p3 = _agg128(h3, src, dst)
    return _final(p3, cntp, h3, batch.reshape(1, N), Wlo, Wro, bo)


# speedup vs baseline: 1.1057x; 1.0866x over previous
import functools

import jax
import jax.numpy as jnp
from jax import lax
from jax.experimental import pallas as pl
from jax.experimental.pallas import tpu as pltpu
from jax.experimental.pallas import tpu_sc as plsc

N = 10000
D = 128
T = 64
G = 64

ACCROWS = 10112
C = 128
NCHUNK = 80
NBUF = 2
NSTAGE = 2
NW = 32
EP = NW * NCHUNK * C
TROWS = ACCROWS // 16

_mesh = plsc.VectorSubcoreMesh(core_axis_name="c", subcore_axis_name="s")


def _make_agg(d):
    @functools.partial(
        pl.kernel,
        out_type=jax.ShapeDtypeStruct((2 * ACCROWS, d), jnp.float32),
        mesh=_mesh,
        scratch_types=[
            pltpu.VMEM((NCHUNK // NSTAGE, C), jnp.int32),
            pltpu.VMEM((NCHUNK // NSTAGE, C), jnp.int32),
            pltpu.VMEM_SHARED((ACCROWS, d), jnp.float32),
        ] + [pltpu.VMEM((C, d), jnp.float32) for _ in range(2)]
          + [pltpu.SemaphoreType.DMA] * 2,
    )
    def agg(table_hbm, src_hbm, dst_hbm, out_hbm, src_v, dst_v, acc, *rest):
        bufa, bufb, sga, sgb = rest
        cid = lax.axis_index("c")
        sid = lax.axis_index("s")
        wid = cid * 16 + sid
        stg = NCHUNK // NSTAGE

        @pl.loop(0, C)
        def _(r):
            @pl.loop(0, d, step=16)
            def _(cc):
                bufa[r, pl.ds(cc, 16)] = jnp.zeros((16,), jnp.float32)

        r0 = sid * TROWS
        nfull = TROWS // C
        for k in range(nfull):
            pltpu.sync_copy(bufa, acc.at[pl.ds(r0 + k * C, C)])
        rem = TROWS - nfull * C
        if rem:
            pltpu.sync_copy(bufa.at[pl.ds(0, rem)],
                            acc.at[pl.ds(r0 + nfull * C, rem)])

        plsc.subcore_barrier()

        def start_gather(j, buf, sem):
            pltpu.make_async_copy(table_hbm.at[src_v.at[j]], buf, sem).start()

        def wait_gather(j, buf, sem):
            pltpu.make_async_copy(table_hbm.at[src_v.at[j]], buf, sem).wait()

        def scatter_add(j, buf):
            pltpu.sync_copy(buf, acc.at[dst_v.at[j]], add=True)

        for p in range(NSTAGE):
            pltpu.sync_copy(src_hbm.at[wid, pl.ds(p * stg, stg)], src_v)
            pltpu.sync_copy(dst_hbm.at[wid, pl.ds(p * stg, stg)], dst_v)
            start_gather(0, bufa, sga)

            @pl.loop(0, stg, step=2)
            def _(j):
                wait_gather(j, bufa, sga)
                start_gather(j + 1, bufb, sgb)
                scatter_add(j, bufa)
                wait_gather(j + 1, bufb, sgb)

                @pl.when(j + 2 < stg)
                def _():
                    start_gather(j + 2, bufa, sga)

                scatter_add(j + 1, bufb)

        plsc.subcore_barrier()

        oo = pl.multiple_of(cid * ACCROWS + r0, 8)
        pltpu.sync_copy(acc.at[pl.ds(r0, TROWS)],
                        out_hbm.at[pl.ds(oo, TROWS)])

    return agg


@functools.partial(
    pl.kernel,
    out_type=jax.ShapeDtypeStruct((2 * ACCROWS, D), jnp.float32),
    mesh=_mesh,
    scratch_types=[
        pltpu.VMEM((NCHUNK, C), jnp.int32),
        pltpu.VMEM((C, D), jnp.float32),
        pltpu.VMEM_SHARED((ACCROWS, D), jnp.float32),
    ],
)
def _cnt_kernel(dst_hbm, out_hbm, dst_v, ones_v, acc):
    cid = lax.axis_index("c")
    sid = lax.axis_index("s")
    wid = cid * 16 + sid
    pltpu.sync_copy(dst_hbm.at[wid], dst_v)

    @pl.loop(0, C)
    def _(r):
        @pl.loop(0, D, step=16)
        def _(cc):
            ones_v[r, pl.ds(cc, 16)] = jnp.zeros((16,), jnp.float32)

    r0 = sid * TROWS
    nfull = TROWS // C
    for k in range(nfull):
        pltpu.sync_copy(ones_v, acc.at[pl.ds(r0 + k * C, C)])
    rem = TROWS - nfull * C
    if rem:
        pltpu.sync_copy(ones_v.at[pl.ds(0, rem)],
                        acc.at[pl.ds(r0 + nfull * C, rem)])

    @pl.loop(0, C)
    def _(r):
        @pl.loop(0, D, step=16)
        def _(cc):
            ones_v[r, pl.ds(cc, 16)] = jnp.ones((16,), jnp.float32)

    plsc.subcore_barrier()

    @pl.loop(0, NCHUNK)
    def _(j):
        pltpu.sync_copy(ones_v, acc.at[dst_v.at[j]], add=True)

    plsc.subcore_barrier()
    oo = pl.multiple_of(cid * ACCROWS + r0, 8)
    pltpu.sync_copy(acc.at[pl.ds(r0, TROWS)], out_hbm.at[pl.ds(oo, TROWS)])


_agg128 = _make_agg(D)


def _mean_from_partials(p_ref, cnt_ref):
    psum = p_ref[0:N, :] + p_ref[ACCROWS:ACCROWS + N, :]
    cnt = cnt_ref[0:N, 0:1] + cnt_ref[ACCROWS:ACCROWS + N, 0:1]
    return psum / jnp.maximum(cnt, 1.0)


def _dense_body(p_ref, cnt_ref, h_ref, wl_ref, wr_ref, b_ref, g_ref, be_ref,
                o_ref):
    mean = _mean_from_partials(p_ref, cnt_ref)
    z = (jnp.dot(mean, wl_ref[...], preferred_element_type=jnp.float32)
         + jnp.dot(h_ref[...], wr_ref[...], preferred_element_type=jnp.float32)
         + b_ref[...])
    mu = jnp.mean(z, axis=0, keepdims=True)
    var = jnp.mean((z - mu) * (z - mu), axis=0, keepdims=True)
    zn = (z - mu) / jnp.sqrt(var + 1e-5) * g_ref[...] + be_ref[...]
    o_ref[...] = jnp.maximum(zn, 0.0)


def _final_body(p_ref, cnt_ref, h_ref, batch_ref, wlo_ref, wro_ref, bo_ref,
                o_ref):
    s = _mean_from_partials(p_ref, cnt_ref)
    gi = lax.broadcasted_iota(jnp.int32, (G, N), 0)
    bm = (batch_ref[0:1, :] == gi).astype(jnp.float32)
    cg = jnp.sum(bm, axis=1, keepdims=True)
    pw = bm * jnp.where(cg > 0, 1.0 / jnp.maximum(cg, 1.0), 0.0)
    pooled_s = jnp.dot(pw, s, preferred_element_type=jnp.float32)
    pooled_h = jnp.dot(pw, h_ref[...], preferred_element_type=jnp.float32)
    o = (jnp.dot(pooled_s, wlo_ref[...], preferred_element_type=jnp.float32)
         + jnp.dot(pooled_h, wro_ref[...], preferred_element_type=jnp.float32)
         + bo_ref[...])
    o_ref[...] = jnp.where(cg > 0, o, 0.0)


def _dense(p, cntp, h, wl, wr, b, g, be):
    return pl.pallas_call(
        _dense_body,
        out_shape=jax.ShapeDtypeStruct((N, D), jnp.float32),
    )(p, cntp, h, wl, wr, b.reshape(1, -1), g.reshape(1, -1),
      be.reshape(1, -1))


def _final(p, cntp, h, batch2d, wlo, wro, bo):
    return pl.pallas_call(
        _final_body,
        out_shape=jax.ShapeDtypeStruct((G, T), jnp.float32),
    )(p, cntp, h, batch2d, wlo, wro, bo.reshape(1, -1))


def kernel(x, edge_index, batch, Wl0, Wr0, b0, g1, be1, Wl1, Wr1, b1, g2, be2,
           Wl2, Wr2, b2, g3, be3, Wlo, Wro, bo):
    e = edge_index.shape[1]
    pad = EP - e
    pidx = jnp.arange(pad, dtype=jnp.int32)
    src = jnp.concatenate(
        [edge_index[0], pidx % N]).reshape(NW, NCHUNK, C)
    dst = jnp.concatenate(
        [edge_index[1], N + pidx % (ACCROWS - N)]).reshape(NW, NCHUNK, C)

    cntp = _cnt_kernel(dst)
    p0 = _agg128(x, src, dst)
    h1 = _dense(p0, cntp, x, Wl0, Wr0, b0, g1, be1)
    p1 = _agg128(h1, src, dst)
    h2 = _dense(p1, cntp, h1, Wl1, Wr1, b1, g2, be2)
    p2 = _agg128(h2, src, dst)
    h3 = _dense(p2, cntp, h2, Wl2, Wr2, b2, g3, be3)
    p3 = _agg128(h3, src, dst)
    return _final(p3, cntp, h3, batch.reshape(1, N), Wlo, Wro, bo)


# speedup vs baseline: 1.1160x; 1.0094x over previous
import functools

import jax
import jax.numpy as jnp
from jax import lax
from jax.experimental import pallas as pl
from jax.experimental.pallas import tpu as pltpu
from jax.experimental.pallas import tpu_sc as plsc

N = 10000
D = 128
T = 64
G = 64

ACCROWS = 10112
C = 128
NCHUNK = 80
NBUF = 2
NSTAGE = 2
NW = 32
EP = NW * NCHUNK * C
TROWS = ACCROWS // 16

_mesh = plsc.VectorSubcoreMesh(core_axis_name="c", subcore_axis_name="s")


def _make_agg(d):
    @functools.partial(
        pl.kernel,
        out_type=jax.ShapeDtypeStruct((2 * ACCROWS, d), jnp.float32),
        mesh=_mesh,
        scratch_types=[
            pltpu.VMEM((NCHUNK // NSTAGE, C), jnp.int32),
            pltpu.VMEM((NCHUNK // NSTAGE, C), jnp.int32),
            pltpu.VMEM_SHARED((ACCROWS, d), jnp.float32),
        ] + [pltpu.VMEM((C, d), jnp.float32) for _ in range(2)]
          + [pltpu.SemaphoreType.DMA] * 2,
    )
    def agg(table_hbm, src_hbm, dst_hbm, out_hbm, src_v, dst_v, acc, *rest):
        bufa, bufb, sga, sgb = rest
        cid = lax.axis_index("c")
        sid = lax.axis_index("s")
        wid = cid * 16 + sid
        stg = NCHUNK // NSTAGE

        @pl.loop(0, C)
        def _(r):
            @pl.loop(0, d, step=16)
            def _(cc):
                bufa[r, pl.ds(cc, 16)] = jnp.zeros((16,), jnp.float32)

        r0 = sid * TROWS
        nfull = TROWS // C
        for k in range(nfull):
            pltpu.sync_copy(bufa, acc.at[pl.ds(r0 + k * C, C)])
        rem = TROWS - nfull * C
        if rem:
            pltpu.sync_copy(bufa.at[pl.ds(0, rem)],
                            acc.at[pl.ds(r0 + nfull * C, rem)])

        plsc.subcore_barrier()

        def start_gather(j, buf, sem):
            pltpu.make_async_copy(table_hbm.at[src_v.at[j]], buf, sem).start()

        def wait_gather(j, buf, sem):
            pltpu.make_async_copy(table_hbm.at[src_v.at[j]], buf, sem).wait()

        def scatter_add(j, buf):
            pltpu.sync_copy(buf, acc.at[dst_v.at[j]], add=True)

        for p in range(NSTAGE):
            pltpu.sync_copy(src_hbm.at[wid, pl.ds(p * stg, stg)], src_v)
            pltpu.sync_copy(dst_hbm.at[wid, pl.ds(p * stg, stg)], dst_v)
            start_gather(0, bufa, sga)

            @pl.loop(0, stg, step=2)
            def _(j):
                wait_gather(j, bufa, sga)
                start_gather(j + 1, bufb, sgb)
                scatter_add(j, bufa)
                wait_gather(j + 1, bufb, sgb)

                @pl.when(j + 2 < stg)
                def _():
                    start_gather(j + 2, bufa, sga)

                scatter_add(j + 1, bufb)

        plsc.subcore_barrier()

        oo = pl.multiple_of(cid * ACCROWS + r0, 8)
        pltpu.sync_copy(acc.at[pl.ds(r0, TROWS)],
                        out_hbm.at[pl.ds(oo, TROWS)])

    return agg


@functools.partial(
    pl.kernel,
    out_type=jax.ShapeDtypeStruct((2 * ACCROWS, D), jnp.float32),
    mesh=_mesh,
    scratch_types=[
        pltpu.VMEM((NCHUNK, C), jnp.int32),
        pltpu.VMEM((C, D), jnp.float32),
        pltpu.VMEM_SHARED((ACCROWS, D), jnp.float32),
    ],
)
def _cnt_kernel(dst_hbm, out_hbm, dst_v, ones_v, acc):
    cid = lax.axis_index("c")
    sid = lax.axis_index("s")
    wid = cid * 16 + sid
    pltpu.sync_copy(dst_hbm.at[wid], dst_v)

    @pl.loop(0, C)
    def _(r):
        @pl.loop(0, D, step=16)
        def _(cc):
            ones_v[r, pl.ds(cc, 16)] = jnp.zeros((16,), jnp.float32)

    r0 = sid * TROWS
    nfull = TROWS // C
    for k in range(nfull):
        pltpu.sync_copy(ones_v, acc.at[pl.ds(r0 + k * C, C)])
    rem = TROWS - nfull * C
    if rem:
        pltpu.sync_copy(ones_v.at[pl.ds(0, rem)],
                        acc.at[pl.ds(r0 + nfull * C, rem)])

    @pl.loop(0, C)
    def _(r):
        @pl.loop(0, D, step=16)
        def _(cc):
            ones_v[r, pl.ds(cc, 16)] = jnp.ones((16,), jnp.float32)

    plsc.subcore_barrier()

    @pl.loop(0, NCHUNK)
    def _(j):
        pltpu.sync_copy(ones_v, acc.at[dst_v.at[j]], add=True)

    plsc.subcore_barrier()
    oo = pl.multiple_of(cid * ACCROWS + r0, 8)
    pltpu.sync_copy(acc.at[pl.ds(r0, TROWS)], out_hbm.at[pl.ds(oo, TROWS)])


_agg128 = _make_agg(D)


def _psum(p_ref):
    return p_ref[0:N, :] + p_ref[ACCROWS:ACCROWS + N, :]


def _bn_relu(z, g, be):
    mu = jnp.mean(z, axis=0, keepdims=True)
    var = jnp.mean((z - mu) * (z - mu), axis=0, keepdims=True)
    return jnp.maximum((z - mu) / jnp.sqrt(var + 1e-5) * g + be, 0.0)


def _poolw(batch_ref):
    gi = lax.broadcasted_iota(jnp.int32, (G, N), 0)
    bm = (batch_ref[0:1, :] == gi).astype(jnp.float32)
    cg = jnp.sum(bm, axis=1, keepdims=True)
    pw = bm * jnp.where(cg > 0, 1.0 / jnp.maximum(cg, 1.0), 0.0)
    return pw, cg


def _pre_body(h_ref, w_ref, b_ref, o_ref):
    o_ref[...] = (jnp.dot(h_ref[...], w_ref[...],
                          preferred_element_type=jnp.float32) + b_ref[...])


def _post0_body(p_ref, cnt_ref, pre_ref, wl_ref, g_ref, be_ref, o_ref,
                inv_ref):
    cnt = cnt_ref[0:N, 0:1] + cnt_ref[ACCROWS:ACCROWS + N, 0:1]
    inv = 1.0 / jnp.maximum(cnt, 1.0)
    inv_ref[...] = inv
    z = (jnp.dot(_psum(p_ref) * inv, wl_ref[...],
                 preferred_element_type=jnp.float32) + pre_ref[...])
    o_ref[...] = _bn_relu(z, g_ref[...], be_ref[...])


def _post_body(p_ref, inv_ref, pre_ref, wl_ref, g_ref, be_ref, o_ref):
    z = (jnp.dot(_psum(p_ref) * inv_ref[...], wl_ref[...],
                 preferred_element_type=jnp.float32) + pre_ref[...])
    o_ref[...] = _bn_relu(z, g_ref[...], be_ref[...])


def _prefinal_body(h_ref, batch_ref, wro_ref, bo_ref, o_ref):
    pw, _ = _poolw(batch_ref)
    ph = jnp.dot(pw, h_ref[...], preferred_element_type=jnp.float32)
    o_ref[...] = (jnp.dot(ph, wro_ref[...],
                          preferred_element_type=jnp.float32) + bo_ref[...])


def _final_body(p_ref, inv_ref, batch_ref, wlo_ref, pf_ref, o_ref):
    s = _psum(p_ref) * inv_ref[...]
    pw, cg = _poolw(batch_ref)
    ps = jnp.dot(pw, s, preferred_element_type=jnp.float32)
    o = (jnp.dot(ps, wlo_ref[...], preferred_element_type=jnp.float32)
         + pf_ref[...])
    o_ref[...] = jnp.where(cg > 0, o, 0.0)


def _pre(h, w, b):
    return pl.pallas_call(
        _pre_body,
        out_shape=jax.ShapeDtypeStruct((N, D), jnp.float32),
    )(h, w, b.reshape(1, -1))


def _post0(p, cntp, pre, wl, g, be):
    return pl.pallas_call(
        _post0_body,
        out_shape=[jax.ShapeDtypeStruct((N, D), jnp.float32),
                   jax.ShapeDtypeStruct((N, 1), jnp.float32)],
    )(p, cntp, pre, wl, g.reshape(1, -1), be.reshape(1, -1))


def _post(p, invc, pre, wl, g, be):
    return pl.pallas_call(
        _post_body,
        out_shape=jax.ShapeDtypeStruct((N, D), jnp.float32),
    )(p, invc, pre, wl, g.reshape(1, -1), be.reshape(1, -1))


def _prefinal(h, batch2d, wro, bo):
    return pl.pallas_call(
        _prefinal_body,
        out_shape=jax.ShapeDtypeStruct((G, T), jnp.float32),
    )(h, batch2d, wro, bo.reshape(1, -1))


def _final(p, invc, batch2d, wlo, pf):
    return pl.pallas_call(
        _final_body,
        out_shape=jax.ShapeDtypeStruct((G, T), jnp.float32),
    )(p, invc, batch2d, wlo, pf)


def kernel(x, edge_index, batch, Wl0, Wr0, b0, g1, be1, Wl1, Wr1, b1, g2, be2,
           Wl2, Wr2, b2, g3, be3, Wlo, Wro, bo):
    e = edge_index.shape[1]
    pad = EP - e
    pidx = jnp.arange(pad, dtype=jnp.int32)
    src = jnp.concatenate(
        [edge_index[0], pidx % N]).reshape(NW, NCHUNK, C)
    dst = jnp.concatenate(
        [edge_index[1], N + pidx % (ACCROWS - N)]).reshape(NW, NCHUNK, C)
    batch2d = batch.reshape(1, N)

    cntp = _cnt_kernel(dst)
    pre0 = _pre(x, Wr0, b0)
    p0 = _agg128(x, src, dst)
    h1, invc = _post0(p0, cntp, pre0, Wl0, g1, be1)
    pre1 = _pre(h1, Wr1, b1)
    p1 = _agg128(h1, src, dst)
    h2 = _post(p1, invc, pre1, Wl1, g2, be2)
    pre2 = _pre(h2, Wr2, b2)
    p2 = _agg128(h2, src, dst)
    h3 = _post(p2, invc, pre2, Wl2, g3, be3)
    pf = _prefinal(h3, batch2d, Wro, bo)
    p3 = _agg128(h3, src, dst)
    return _final(p3, invc, batch2d, Wlo, pf)


# speedup vs baseline: 1.1318x; 1.0142x over previous
import functools

import jax
import jax.numpy as jnp
from jax import lax
from jax.experimental import pallas as pl
from jax.experimental.pallas import tpu as pltpu
from jax.experimental.pallas import tpu_sc as plsc

N = 10000
D = 128
T = 64
G = 64

ACCROWS = 10112
C = 128
NCHUNK = 80
NBUF = 2
NSTAGE = 2
NW = 32
EP = NW * NCHUNK * C
TROWS = ACCROWS // 16

_mesh = plsc.VectorSubcoreMesh(core_axis_name="c", subcore_axis_name="s")


def _make_agg(d):
    @functools.partial(
        pl.kernel,
        out_type=jax.ShapeDtypeStruct((2 * ACCROWS, d), jnp.float32),
        mesh=_mesh,
        scratch_types=[
            pltpu.VMEM((NCHUNK // NSTAGE, C), jnp.int32),
            pltpu.VMEM((NCHUNK // NSTAGE, C), jnp.int32),
            pltpu.VMEM_SHARED((ACCROWS, d), jnp.float32),
        ] + [pltpu.VMEM((C, d), jnp.float32) for _ in range(2)]
          + [pltpu.SemaphoreType.DMA] * 2,
    )
    def agg(table_hbm, ei_hbm, out_hbm, src_v, dst_v, acc, *rest):
        bufa, bufb, sga, sgb = rest
        cid = lax.axis_index("c")
        sid = lax.axis_index("s")
        wid = cid * 16 + sid
        stg = NCHUNK // NSTAGE

        @pl.loop(0, C)
        def _(r):
            @pl.loop(0, d, step=16)
            def _(cc):
                bufa[r, pl.ds(cc, 16)] = jnp.zeros((16,), jnp.float32)

        r0 = sid * TROWS
        nfull = TROWS // C
        for k in range(nfull):
            pltpu.sync_copy(bufa, acc.at[pl.ds(r0 + k * C, C)])
        rem = TROWS - nfull * C
        if rem:
            pltpu.sync_copy(bufa.at[pl.ds(0, rem)],
                            acc.at[pl.ds(r0 + nfull * C, rem)])

        plsc.subcore_barrier()

        def start_gather(j, buf, sem):
            pltpu.make_async_copy(table_hbm.at[src_v.at[j]], buf, sem).start()

        def wait_gather(j, buf, sem):
            pltpu.make_async_copy(table_hbm.at[src_v.at[j]], buf, sem).wait()

        def scatter_add(j, buf):
            pltpu.sync_copy(buf, acc.at[dst_v.at[j]], add=True)

        for p in range(NSTAGE):
            cb = wid * NCHUNK + p * stg
            pltpu.sync_copy(ei_hbm.at[0, pl.ds(cb, stg)], src_v)
            pltpu.sync_copy(ei_hbm.at[1, pl.ds(cb, stg)], dst_v)
            start_gather(0, bufa, sga)

            @pl.loop(0, stg, step=2)
            def _(j):
                wait_gather(j, bufa, sga)
                start_gather(j + 1, bufb, sgb)
                scatter_add(j, bufa)
                wait_gather(j + 1, bufb, sgb)

                @pl.when(j + 2 < stg)
                def _():
                    start_gather(j + 2, bufa, sga)

                scatter_add(j + 1, bufb)

        plsc.subcore_barrier()

        oo = pl.multiple_of(cid * ACCROWS + r0, 8)
        pltpu.sync_copy(acc.at[pl.ds(r0, TROWS)],
                        out_hbm.at[pl.ds(oo, TROWS)])

    return agg


@functools.partial(
    pl.kernel,
    out_type=jax.ShapeDtypeStruct((2 * ACCROWS, D), jnp.float32),
    mesh=_mesh,
    scratch_types=[
        pltpu.VMEM((NCHUNK, C), jnp.int32),
        pltpu.VMEM((C, D), jnp.float32),
        pltpu.VMEM_SHARED((ACCROWS, D), jnp.float32),
    ],
)
def _cnt_kernel(ei_hbm, out_hbm, dst_v, ones_v, acc):
    cid = lax.axis_index("c")
    sid = lax.axis_index("s")
    wid = cid * 16 + sid
    pltpu.sync_copy(ei_hbm.at[1, pl.ds(wid * NCHUNK, NCHUNK)], dst_v)

    @pl.loop(0, C)
    def _(r):
        @pl.loop(0, D, step=16)
        def _(cc):
            ones_v[r, pl.ds(cc, 16)] = jnp.zeros((16,), jnp.float32)

    r0 = sid * TROWS
    nfull = TROWS // C
    for k in range(nfull):
        pltpu.sync_copy(ones_v, acc.at[pl.ds(r0 + k * C, C)])
    rem = TROWS - nfull * C
    if rem:
        pltpu.sync_copy(ones_v.at[pl.ds(0, rem)],
                        acc.at[pl.ds(r0 + nfull * C, rem)])

    @pl.loop(0, C)
    def _(r):
        @pl.loop(0, D, step=16)
        def _(cc):
            ones_v[r, pl.ds(cc, 16)] = jnp.ones((16,), jnp.float32)

    plsc.subcore_barrier()

    @pl.loop(0, NCHUNK)
    def _(j):
        pltpu.sync_copy(ones_v, acc.at[dst_v.at[j]], add=True)

    plsc.subcore_barrier()
    oo = pl.multiple_of(cid * ACCROWS + r0, 8)
    pltpu.sync_copy(acc.at[pl.ds(r0, TROWS)], out_hbm.at[pl.ds(oo, TROWS)])


_agg128 = _make_agg(D)


def _psum(p_ref):
    return p_ref[0:N, :] + p_ref[ACCROWS:ACCROWS + N, :]


def _bn_relu(z, g, be):
    mu = jnp.mean(z, axis=0, keepdims=True)
    var = jnp.mean((z - mu) * (z - mu), axis=0, keepdims=True)
    return jnp.maximum((z - mu) / jnp.sqrt(var + 1e-5) * g + be, 0.0)


def _poolw(batch_ref):
    gi = lax.broadcasted_iota(jnp.int32, (G, N), 0)
    bm = (batch_ref[0:1, :] == gi).astype(jnp.float32)
    cg = jnp.sum(bm, axis=1, keepdims=True)
    pw = bm * jnp.where(cg > 0, 1.0 / jnp.maximum(cg, 1.0), 0.0)
    return pw, cg


def _pre_body(h_ref, w_ref, b_ref, o_ref):
    o_ref[...] = (jnp.dot(h_ref[...], w_ref[...],
                          preferred_element_type=jnp.float32) + b_ref[...])


def _post0_body(p_ref, cnt_ref, pre_ref, wl_ref, g_ref, be_ref, o_ref,
                inv_ref):
    cnt = cnt_ref[0:N, 0:1] + cnt_ref[ACCROWS:ACCROWS + N, 0:1]
    inv = 1.0 / jnp.maximum(cnt, 1.0)
    inv_ref[...] = inv
    z = (jnp.dot(_psum(p_ref) * inv, wl_ref[...],
                 preferred_element_type=jnp.float32) + pre_ref[...])
    o_ref[...] = _bn_relu(z, g_ref[...], be_ref[...])


def _post_body(p_ref, inv_ref, pre_ref, wl_ref, g_ref, be_ref, o_ref):
    z = (jnp.dot(_psum(p_ref) * inv_ref[...], wl_ref[...],
                 preferred_element_type=jnp.float32) + pre_ref[...])
    o_ref[...] = _bn_relu(z, g_ref[...], be_ref[...])


def _prefinal_body(h_ref, batch_ref, wro_ref, bo_ref, o_ref):
    pw, _ = _poolw(batch_ref)
    ph = jnp.dot(pw, h_ref[...], preferred_element_type=jnp.float32)
    o_ref[...] = (jnp.dot(ph, wro_ref[...],
                          preferred_element_type=jnp.float32) + bo_ref[...])


def _final_body(p_ref, inv_ref, batch_ref, wlo_ref, pf_ref, o_ref):
    s = _psum(p_ref) * inv_ref[...]
    pw, cg = _poolw(batch_ref)
    ps = jnp.dot(pw, s, preferred_element_type=jnp.float32)
    o = (jnp.dot(ps, wlo_ref[...], preferred_element_type=jnp.float32)
         + pf_ref[...])
    o_ref[...] = jnp.where(cg > 0, o, 0.0)


def _pre(h, w, b):
    return pl.pallas_call(
        _pre_body,
        out_shape=jax.ShapeDtypeStruct((N, D), jnp.float32),
    )(h, w, b.reshape(1, -1))


def _post0(p, cntp, pre, wl, g, be):
    return pl.pallas_call(
        _post0_body,
        out_shape=[jax.ShapeDtypeStruct((N, D), jnp.float32),
                   jax.ShapeDtypeStruct((N, 1), jnp.float32)],
    )(p, cntp, pre, wl, g.reshape(1, -1), be.reshape(1, -1))


def _post(p, invc, pre, wl, g, be):
    return pl.pallas_call(
        _post_body,
        out_shape=jax.ShapeDtypeStruct((N, D), jnp.float32),
    )(p, invc, pre, wl, g.reshape(1, -1), be.reshape(1, -1))


def _prefinal(h, batch2d, wro, bo):
    return pl.pallas_call(
        _prefinal_body,
        out_shape=jax.ShapeDtypeStruct((G, T), jnp.float32),
    )(h, batch2d, wro, bo.reshape(1, -1))


def _final(p, invc, batch2d, wlo, pf):
    return pl.pallas_call(
        _final_body,
        out_shape=jax.ShapeDtypeStruct((G, T), jnp.float32),
    )(p, invc, batch2d, wlo, pf)


def kernel(x, edge_index, batch, Wl0, Wr0, b0, g1, be1, Wl1, Wr1, b1, g2, be2,
           Wl2, Wr2, b2, g3, be3, Wlo, Wro, bo):
    e = edge_index.shape[1]
    nreal = e // C
    npadc = NW * NCHUNK - nreal
    pidx = jnp.arange(npadc * C, dtype=jnp.int32).reshape(npadc, C)
    padc = jnp.stack([pidx % N, N + pidx % (ACCROWS - N)])
    ei = jnp.concatenate([edge_index.reshape(2, nreal, C), padc], axis=1)
    batch2d = batch.reshape(1, N)

    cntp = _cnt_kernel(ei)
    pre0 = _pre(x, Wr0, b0)
    p0 = _agg128(x, ei)
    h1, invc = _post0(p0, cntp, pre0, Wl0, g1, be1)
    pre1 = _pre(h1, Wr1, b1)
    p1 = _agg128(h1, ei)
    h2 = _post(p1, invc, pre1, Wl1, g2, be2)
    pre2 = _pre(h2, Wr2, b2)
    p2 = _agg128(h2, ei)
    h3 = _post(p2, invc, pre2, Wl2, g3, be3)
    pf = _prefinal(h3, batch2d, Wro, bo)
    p3 = _agg128(h3, ei)
    return _final(p3, invc, batch2d, Wlo, pf)
